# baseline (speedup 1.0000x reference)
import functools

import jax
import jax.numpy as jnp
from jax import lax
from jax.experimental import pallas as pl
from jax.experimental.pallas import tpu as pltpu
from jax.experimental.pallas import tpu_sc as plsc

NUM_USERS = 2000
NUM_ITEMS = 8000
EMBED_DIM = 128
N_LAYERS = 3
N_NODES = NUM_USERS + NUM_ITEMS
N_EDGES = 320000

NC = 2
NS = 16
L = 16
NW = NC * NS

CHUNK = 64
CPT = 160
PAD_E = NW * CPT * CHUNK

ROWS_PER_SUB = 624
TAIL_ROWS = N_NODES - ROWS_PER_SUB * NS
ZROWS = 16


def _sc_layer(table, srcp, dstp, wp):
  mesh = plsc.VectorSubcoreMesh(core_axis_name="c", subcore_axis_name="s")

  @functools.partial(
      pl.kernel,
      out_type=jax.ShapeDtypeStruct((NC, N_NODES, EMBED_DIM), jnp.float32),
      mesh=mesh,
      scratch_types=[
          pltpu.VMEM((CPT * CHUNK,), jnp.int32),
          pltpu.VMEM((CPT, CHUNK), jnp.int32),
          pltpu.VMEM((CHUNK,), jnp.float32),
          pltpu.VMEM((CHUNK,), jnp.float32),
          pltpu.VMEM((CHUNK, EMBED_DIM), jnp.float32),
          pltpu.VMEM((CHUNK, EMBED_DIM), jnp.float32),
          pltpu.VMEM((ZROWS, EMBED_DIM), jnp.float32),
          pltpu.VMEM_SHARED((N_NODES, EMBED_DIM), jnp.float32),
          pltpu.SemaphoreType.DMA,
          pltpu.SemaphoreType.DMA,
          pltpu.SemaphoreType.DMA,
          pltpu.SemaphoreType.DMA,
      ],
  )
  def k(table_h, src_h, dst_h, w_h, out_h, src_all, dst_all, w0, w1, rows0,
        rows1, zero_v, acc_sh, sem0, sem1, sem2, sem3):
    c = lax.axis_index("c")
    s = lax.axis_index("s")
    wid = s * NC + c
    eb = wid * (CPT * CHUNK)

    def dpre(i, _):
      o = pl.ds(eb + i * CHUNK, CHUNK)
      v = pl.ds(i * CHUNK, CHUNK)
      pltpu.async_copy(src_h.at[o], src_all.at[v], sem1)
      pltpu.async_copy(dst_h.at[o], dst_all.at[i], sem1)
      return 0

    lax.fori_loop(0, CPT, dpre, 0)

    zeros16 = jnp.zeros((L,), jnp.float32)

    def zbody(i, _):
      for d in range(EMBED_DIM // L):
        zero_v[i, pl.ds(d * L, L)] = zeros16
      return 0

    lax.fori_loop(0, ZROWS, zbody, 0)
    for z in range(ROWS_PER_SUB // ZROWS):
      pltpu.sync_copy(zero_v,
                      acc_sh.at[pl.ds(s * ROWS_PER_SUB + z * ZROWS, ZROWS)])

    @pl.when(s == 0)
    def _():
      pltpu.sync_copy(zero_v.at[pl.ds(0, TAIL_ROWS)],
                      acc_sh.at[pl.ds(ROWS_PER_SUB * NS, TAIL_ROWS)])

    def ddrain(i, _):
      o = pl.ds(eb + i * CHUNK, CHUNK)
      v = pl.ds(i * CHUNK, CHUNK)
      pltpu.make_async_copy(src_h.at[o], src_all.at[v], sem1).wait()
      pltpu.make_async_copy(dst_h.at[o], dst_all.at[i], sem1).wait()
      return 0

    lax.fori_loop(0, CPT, ddrain, 0)
    plsc.subcore_barrier()

    rows = (rows0, rows1)
    wbufs = (w0, w1)
    gsems = (sem0, sem1)
    ssems = (sem2, sem3)

    def gather_start(ci, b):
      pltpu.async_copy(w_h.at[pl.ds(eb + ci * CHUNK, CHUNK)], wbufs[b],
                       gsems[b])
      pltpu.async_copy(table_h.at[src_all.at[pl.ds(ci * CHUNK, CHUNK)]],
                       rows[b], gsems[b])

    def gather_wait(ci, b):
      pltpu.make_async_copy(w_h.at[pl.ds(eb + ci * CHUNK, CHUNK)], wbufs[b],
                            gsems[b]).wait()
      pltpu.make_async_copy(table_h.at[src_all.at[pl.ds(ci * CHUNK, CHUNK)]],
                            rows[b], gsems[b]).wait()

    def scale(ci, b):
      rv = rows[b]
      wv = wbufs[b]

      def sbody(g, _):
        wg = wv[pl.ds(g * L, L)]
        for j in range(L):
          e = g * L + j
          wsp = jnp.full((L,), wg[j], jnp.float32)
          for d in range(EMBED_DIM // L):
            rv[e, pl.ds(d * L, L)] = rv[e, pl.ds(d * L, L)] * wsp
        return 0

      lax.fori_loop(0, CHUNK // L, sbody, 0)

    def scatter_start(ci, b):
      pltpu.async_copy(rows[b], acc_sh.at[dst_all.at[ci]], ssems[b], add=True)

    def scatter_wait(ci, b):
      pltpu.make_async_copy(rows[b], acc_sh.at[dst_all.at[ci]],
                            ssems[b]).wait()

    gather_start(0, 0)
    gather_start(1, 1)
    gather_wait(0, 0)
    scale(0, 0)
    scatter_start(0, 0)
    gather_wait(1, 1)
    scale(1, 1)
    scatter_wait(0, 0)
    gather_start(2, 0)
    scatter_start(1, 1)

    def pair(p, _):
      i0 = 2 * p
      scatter_wait(i0 - 1, 1)
      gather_start(i0 + 1, 1)
      gather_wait(i0, 0)
      scale(i0, 0)
      scatter_start(i0, 0)
      gather_wait(i0 + 1, 1)
      scale(i0 + 1, 1)
      scatter_wait(i0, 0)
      nxt = jnp.minimum(i0 + 2, CPT - 1)
      gather_start(nxt, 0)
      scatter_start(i0 + 1, 1)
      return 0

    lax.fori_loop(1, CPT // 2, pair, 0)
    scatter_wait(CPT - 1, 1)
    gather_wait(CPT - 1, 0)

    plsc.subcore_barrier()
    pltpu.sync_copy(acc_sh.at[pl.ds(s * ROWS_PER_SUB, ROWS_PER_SUB)],
                    out_h.at[c, pl.ds(s * ROWS_PER_SUB, ROWS_PER_SUB)])

    @pl.when(s == 0)
    def _():
      pltpu.sync_copy(acc_sh.at[pl.ds(ROWS_PER_SUB * NS, TAIL_ROWS)],
                      out_h.at[c, pl.ds(ROWS_PER_SUB * NS, TAIL_ROWS)])

  return k(table, srcp, dstp, wp)


def _combine(partials, acc, final):
  scale = 0.25 if final else 1.0
  nb = 10
  blk = N_NODES // nb

  def body(p_ref, a_ref, t_ref, o_ref):
    t = p_ref[0] + p_ref[1]
    t_ref[...] = t
    o_ref[...] = (a_ref[...] + t) * scale

  return pl.pallas_call(
      body,
      grid=(nb,),
      in_specs=[
          pl.BlockSpec((2, blk, EMBED_DIM), lambda i: (0, i, 0)),
          pl.BlockSpec((blk, EMBED_DIM), lambda i: (i, 0)),
      ],
      out_specs=[pl.BlockSpec((blk, EMBED_DIM), lambda i: (i, 0))] * 2,
      out_shape=[jax.ShapeDtypeStruct((N_NODES, EMBED_DIM), jnp.float32)] * 2,
  )(partials, acc)


def kernel(adj_indices, adj_values, user_emb, item_emb):
  all_emb = jnp.concatenate([user_emb, item_emb], axis=0)
  dst = adj_indices[0].astype(jnp.int32)
  src = adj_indices[1].astype(jnp.int32)

  pad = PAD_E - N_EDGES
  fill = jnp.arange(pad, dtype=jnp.int32) % N_NODES
  srcp = jnp.concatenate([src, fill])
  dstp = jnp.concatenate([dst, fill])
  wp = jnp.concatenate([adj_values, jnp.zeros((pad,), jnp.float32)])

  t = all_emb
  acc = all_emb
  for layer in range(N_LAYERS):
    partials = _sc_layer(t, srcp, dstp, wp)
    t, acc = _combine(partials, acc, final=(layer == N_LAYERS - 1))
  return acc[:NUM_USERS], acc[NUM_USERS:]


# speedup vs baseline: 1.0465x; 1.0465x over previous
import functools

import jax
import jax.numpy as jnp
from jax import lax
from jax.experimental import pallas as pl
from jax.experimental.pallas import tpu as pltpu
from jax.experimental.pallas import tpu_sc as plsc

NUM_USERS = 2000
NUM_ITEMS = 8000
EMBED_DIM = 128
N_LAYERS = 3
N_NODES = NUM_USERS + NUM_ITEMS
N_EDGES = 320000

NC = 2
NS = 16
L = 16
NW = NC * NS

CHUNK = 64
CPT = 160
PAD_E = NW * CPT * CHUNK

ROWS_PER_SUB = 624
TAIL_ROWS = N_NODES - ROWS_PER_SUB * NS
ZROWS = 16


def _sc_layer(table, srcp, dstp, wp):
  mesh = plsc.VectorSubcoreMesh(core_axis_name="c", subcore_axis_name="s")

  @functools.partial(
      pl.kernel,
      out_type=jax.ShapeDtypeStruct((NC, N_NODES, EMBED_DIM), jnp.float32),
      mesh=mesh,
      scratch_types=[
          pltpu.VMEM((CPT * CHUNK,), jnp.int32),
          pltpu.VMEM((CPT, CHUNK), jnp.int32),
          pltpu.VMEM((CHUNK,), jnp.float32),
          pltpu.VMEM((CHUNK,), jnp.float32),
          pltpu.VMEM((CHUNK, EMBED_DIM), jnp.float32),
          pltpu.VMEM((CHUNK, EMBED_DIM), jnp.float32),
          pltpu.VMEM((ZROWS, EMBED_DIM), jnp.float32),
          pltpu.VMEM_SHARED((N_NODES, EMBED_DIM), jnp.float32),
          pltpu.SemaphoreType.DMA,
          pltpu.SemaphoreType.DMA,
      ],
  )
  def k(table_h, src_h, dst_h, w_h, out_h, src_all, dst_all, w0, w1, rows0,
        rows1, zero_v, acc_sh, sem0, sem1):
    c = lax.axis_index("c")
    s = lax.axis_index("s")
    wid = s * NC + c
    eb = wid * (CPT * CHUNK)

    def dpre(i, _):
      o = pl.ds(eb + i * CHUNK, CHUNK)
      v = pl.ds(i * CHUNK, CHUNK)
      pltpu.async_copy(src_h.at[o], src_all.at[v], sem1)
      pltpu.async_copy(dst_h.at[o], dst_all.at[i], sem1)
      return 0

    lax.fori_loop(0, CPT, dpre, 0)

    zeros16 = jnp.zeros((L,), jnp.float32)

    def zbody(i, _):
      for d in range(EMBED_DIM // L):
        zero_v[i, pl.ds(d * L, L)] = zeros16
      return 0

    lax.fori_loop(0, ZROWS, zbody, 0)
    for z in range(ROWS_PER_SUB // ZROWS):
      pltpu.sync_copy(zero_v,
                      acc_sh.at[pl.ds(s * ROWS_PER_SUB + z * ZROWS, ZROWS)])

    @pl.when(s == 0)
    def _():
      pltpu.sync_copy(zero_v.at[pl.ds(0, TAIL_ROWS)],
                      acc_sh.at[pl.ds(ROWS_PER_SUB * NS, TAIL_ROWS)])

    def ddrain(i, _):
      o = pl.ds(eb + i * CHUNK, CHUNK)
      v = pl.ds(i * CHUNK, CHUNK)
      pltpu.make_async_copy(src_h.at[o], src_all.at[v], sem1).wait()
      pltpu.make_async_copy(dst_h.at[o], dst_all.at[i], sem1).wait()
      return 0

    lax.fori_loop(0, CPT, ddrain, 0)
    plsc.subcore_barrier()

    rows = (rows0, rows1)
    wbufs = (w0, w1)
    gsems = (sem0, sem1)

    def gather_start(ci, b):
      pltpu.async_copy(w_h.at[pl.ds(eb + ci * CHUNK, CHUNK)], wbufs[b],
                       gsems[b])
      pltpu.async_copy(table_h.at[src_all.at[pl.ds(ci * CHUNK, CHUNK)]],
                       rows[b], gsems[b])

    def gather_wait(ci, b):
      pltpu.make_async_copy(w_h.at[pl.ds(eb + ci * CHUNK, CHUNK)], wbufs[b],
                            gsems[b]).wait()
      pltpu.make_async_copy(table_h.at[src_all.at[pl.ds(ci * CHUNK, CHUNK)]],
                            rows[b], gsems[b]).wait()

    def scale_scatter(ci, b):
      rv = rows[b]
      wv = wbufs[b]

      def sbody(g, _):
        wg = wv[pl.ds(g * L, L)]
        for j in range(L):
          e = g * L + j
          wsp = jnp.full((L,), wg[j], jnp.float32)
          for d in range(EMBED_DIM // L):
            rv[e, pl.ds(d * L, L)] = rv[e, pl.ds(d * L, L)] * wsp
        return 0

      lax.fori_loop(0, CHUNK // L, sbody, 0)
      pltpu.sync_copy(rv, acc_sh.at[dst_all.at[ci]], add=True)

    gather_start(0, 0)

    def pair(p, _):
      i0 = 2 * p
      gather_start(i0 + 1, 1)
      gather_wait(i0, 0)
      scale_scatter(i0, 0)
      nxt = jnp.minimum(i0 + 2, CPT - 1)
      gather_start(nxt, 0)
      gather_wait(i0 + 1, 1)
      scale_scatter(i0 + 1, 1)
      return 0

    lax.fori_loop(0, CPT // 2, pair, 0)
    gather_wait(CPT - 1, 0)

    plsc.subcore_barrier()
    pltpu.sync_copy(acc_sh.at[pl.ds(s * ROWS_PER_SUB, ROWS_PER_SUB)],
                    out_h.at[c, pl.ds(s * ROWS_PER_SUB, ROWS_PER_SUB)])

    @pl.when(s == 0)
    def _():
      pltpu.sync_copy(acc_sh.at[pl.ds(ROWS_PER_SUB * NS, TAIL_ROWS)],
                      out_h.at[c, pl.ds(ROWS_PER_SUB * NS, TAIL_ROWS)])

  return k(table, srcp, dstp, wp)


_NB = 10
_BLK = N_NODES // _NB


def _next_table(partials):
  def body(p_ref, t_ref):
    t_ref[...] = p_ref[0] + p_ref[1]

  return pl.pallas_call(
      body,
      grid=(_NB,),
      in_specs=[pl.BlockSpec((2, _BLK, EMBED_DIM), lambda i: (0, i, 0))],
      out_specs=pl.BlockSpec((_BLK, EMBED_DIM), lambda i: (i, 0)),
      out_shape=jax.ShapeDtypeStruct((N_NODES, EMBED_DIM), jnp.float32),
  )(partials)


def _fold(e0, p1, p2, p3):
  def body(e_ref, a_ref, b_ref, c_ref, o_ref):
    o_ref[...] = (e_ref[...] + (a_ref[0] + a_ref[1]) + (b_ref[0] + b_ref[1]) +
                  (c_ref[0] + c_ref[1])) * 0.25

  pspec = pl.BlockSpec((2, _BLK, EMBED_DIM), lambda i: (0, i, 0))
  espec = pl.BlockSpec((_BLK, EMBED_DIM), lambda i: (i, 0))
  return pl.pallas_call(
      body,
      grid=(_NB,),
      in_specs=[espec, pspec, pspec, pspec],
      out_specs=espec,
      out_shape=jax.ShapeDtypeStruct((N_NODES, EMBED_DIM), jnp.float32),
  )(e0, p1, p2, p3)


def kernel(adj_indices, adj_values, user_emb, item_emb):
  all_emb = jnp.concatenate([user_emb, item_emb], axis=0)
  dst = adj_indices[0].astype(jnp.int32)
  src = adj_indices[1].astype(jnp.int32)

  pad = PAD_E - N_EDGES
  fill = jnp.arange(pad, dtype=jnp.int32) % N_NODES
  srcp = jnp.concatenate([src, fill])
  dstp = jnp.concatenate([dst, fill])
  wp = jnp.concatenate([adj_values, jnp.zeros((pad,), jnp.float32)])

  p1 = _sc_layer(all_emb, srcp, dstp, wp)
  p2 = _sc_layer(_next_table(p1), srcp, dstp, wp)
  p3 = _sc_layer(_next_table(p2), srcp, dstp, wp)
  out = _fold(all_emb, p1, p2, p3)
  return out[:NUM_USERS], out[NUM_USERS:]


# speedup vs baseline: 1.1245x; 1.0746x over previous
import functools

import jax
import jax.numpy as jnp
from jax import lax
from jax.experimental import pallas as pl
from jax.experimental.pallas import tpu as pltpu
from jax.experimental.pallas import tpu_sc as plsc

NUM_USERS = 2000
NUM_ITEMS = 8000
EMBED_DIM = 128
N_LAYERS = 3
N_NODES = NUM_USERS + NUM_ITEMS
N_EDGES = 320000

NC = 2
NS = 16
L = 16
NW = NC * NS

CHUNK = 80
CPT = 128
PAD_E = NW * CPT * CHUNK

ROWS_PER_SUB = 624
TAIL_ROWS = N_NODES - ROWS_PER_SUB * NS


def _sc_layer(table, srcp, dstp, wp):
  mesh = plsc.VectorSubcoreMesh(core_axis_name="c", subcore_axis_name="s")

  @functools.partial(
      pl.kernel,
      out_type=jax.ShapeDtypeStruct((NC, N_NODES, EMBED_DIM), jnp.float32),
      mesh=mesh,
      scratch_types=[
          pltpu.VMEM((CPT * CHUNK,), jnp.int32),
          pltpu.VMEM((CPT, CHUNK), jnp.int32),
          pltpu.VMEM((CHUNK,), jnp.float32),
          pltpu.VMEM((CHUNK,), jnp.float32),
          pltpu.VMEM((CHUNK, EMBED_DIM), jnp.float32),
          pltpu.VMEM((CHUNK, EMBED_DIM), jnp.float32),
          pltpu.VMEM_SHARED((N_NODES, EMBED_DIM), jnp.float32),
          pltpu.SemaphoreType.DMA,
          pltpu.SemaphoreType.DMA,
      ],
  )
  def k(table_h, src_h, dst_h, w_h, out_h, src_all, dst_all, w0, w1, rows0,
        rows1, acc_sh, sem0, sem1):
    c = lax.axis_index("c")
    s = lax.axis_index("s")
    wid = s * NC + c
    eb = wid * (CPT * CHUNK)

    def dpre(i, _):
      o = pl.ds(eb + i * CHUNK, CHUNK)
      v = pl.ds(i * CHUNK, CHUNK)
      pltpu.async_copy(src_h.at[o], src_all.at[v], sem1)
      pltpu.async_copy(dst_h.at[o], dst_all.at[i], sem1)
      return 0

    lax.fori_loop(0, CPT, dpre, 0)

    zeros16 = jnp.zeros((L,), jnp.float32)

    def zbody(i, _):
      for d in range(EMBED_DIM // L):
        rows0[i, pl.ds(d * L, L)] = zeros16
      return 0

    lax.fori_loop(0, CHUNK, zbody, 0)
    for z in range(ROWS_PER_SUB // CHUNK):
      pltpu.sync_copy(rows0,
                      acc_sh.at[pl.ds(s * ROWS_PER_SUB + z * CHUNK, CHUNK)])
    ztail = ROWS_PER_SUB - (ROWS_PER_SUB // CHUNK) * CHUNK
    if ztail:
      pltpu.sync_copy(
          rows0.at[pl.ds(0, ztail)],
          acc_sh.at[pl.ds(s * ROWS_PER_SUB + ROWS_PER_SUB - ztail, ztail)])

    @pl.when(s == 0)
    def _():
      pltpu.sync_copy(rows0.at[pl.ds(0, TAIL_ROWS)],
                      acc_sh.at[pl.ds(ROWS_PER_SUB * NS, TAIL_ROWS)])

    def ddrain(i, _):
      o = pl.ds(eb + i * CHUNK, CHUNK)
      v = pl.ds(i * CHUNK, CHUNK)
      pltpu.make_async_copy(src_h.at[o], src_all.at[v], sem1).wait()
      pltpu.make_async_copy(dst_h.at[o], dst_all.at[i], sem1).wait()
      return 0

    lax.fori_loop(0, CPT, ddrain, 0)
    plsc.subcore_barrier()

    rows = (rows0, rows1)
    wbufs = (w0, w1)
    gsems = (sem0, sem1)

    def gather_start(ci, b):
      pltpu.async_copy(w_h.at[pl.ds(eb + ci * CHUNK, CHUNK)], wbufs[b],
                       gsems[b])
      pltpu.async_copy(table_h.at[src_all.at[pl.ds(ci * CHUNK, CHUNK)]],
                       rows[b], gsems[b])

    def gather_wait(ci, b):
      pltpu.make_async_copy(w_h.at[pl.ds(eb + ci * CHUNK, CHUNK)], wbufs[b],
                            gsems[b]).wait()
      pltpu.make_async_copy(table_h.at[src_all.at[pl.ds(ci * CHUNK, CHUNK)]],
                            rows[b], gsems[b]).wait()

    def scale_scatter(ci, b):
      rv = rows[b]
      wv = wbufs[b]

      def sbody(g, _):
        wg = wv[pl.ds(g * L, L)]
        for j in range(L):
          e = g * L + j
          wsp = jnp.full((L,), wg[j], jnp.float32)
          for d in range(EMBED_DIM // L):
            rv[e, pl.ds(d * L, L)] = rv[e, pl.ds(d * L, L)] * wsp
        return 0

      lax.fori_loop(0, CHUNK // L, sbody, 0)
      pltpu.sync_copy(rv, acc_sh.at[dst_all.at[ci]], add=True)

    gather_start(0, 0)

    def pair(p, _):
      i0 = 2 * p
      gather_start(i0 + 1, 1)
      gather_wait(i0, 0)
      scale_scatter(i0, 0)
      nxt = jnp.minimum(i0 + 2, CPT - 1)
      gather_start(nxt, 0)
      gather_wait(i0 + 1, 1)
      scale_scatter(i0 + 1, 1)
      return 0

    lax.fori_loop(0, CPT // 2, pair, 0)
    gather_wait(CPT - 1, 0)

    plsc.subcore_barrier()
    pltpu.sync_copy(acc_sh.at[pl.ds(s * ROWS_PER_SUB, ROWS_PER_SUB)],
                    out_h.at[c, pl.ds(s * ROWS_PER_SUB, ROWS_PER_SUB)])

    @pl.when(s == 0)
    def _():
      pltpu.sync_copy(acc_sh.at[pl.ds(ROWS_PER_SUB * NS, TAIL_ROWS)],
                      out_h.at[c, pl.ds(ROWS_PER_SUB * NS, TAIL_ROWS)])

  return k(table, srcp, dstp, wp)


_NB = 10
_BLK = N_NODES // _NB


def _next_table(partials):
  def body(p_ref, t_ref):
    t_ref[...] = p_ref[0] + p_ref[1]

  return pl.pallas_call(
      body,
      grid=(_NB,),
      in_specs=[pl.BlockSpec((2, _BLK, EMBED_DIM), lambda i: (0, i, 0))],
      out_specs=pl.BlockSpec((_BLK, EMBED_DIM), lambda i: (i, 0)),
      out_shape=jax.ShapeDtypeStruct((N_NODES, EMBED_DIM), jnp.float32),
  )(partials)


def _fold(e0, p1, p2, p3):
  def body(e_ref, a_ref, b_ref, c_ref, o_ref):
    o_ref[...] = (e_ref[...] + (a_ref[0] + a_ref[1]) + (b_ref[0] + b_ref[1]) +
                  (c_ref[0] + c_ref[1])) * 0.25

  pspec = pl.BlockSpec((2, _BLK, EMBED_DIM), lambda i: (0, i, 0))
  espec = pl.BlockSpec((_BLK, EMBED_DIM), lambda i: (i, 0))
  return pl.pallas_call(
      body,
      grid=(_NB,),
      in_specs=[espec, pspec, pspec, pspec],
      out_specs=espec,
      out_shape=jax.ShapeDtypeStruct((N_NODES, EMBED_DIM), jnp.float32),
  )(e0, p1, p2, p3)


def kernel(adj_indices, adj_values, user_emb, item_emb):
  all_emb = jnp.concatenate([user_emb, item_emb], axis=0)
  dst = adj_indices[0].astype(jnp.int32)
  src = adj_indices[1].astype(jnp.int32)

  pad = PAD_E - N_EDGES
  fill = jnp.arange(pad, dtype=jnp.int32) % N_NODES
  srcp = jnp.concatenate([src, fill])
  dstp = jnp.concatenate([dst, fill])
  wp = jnp.concatenate([adj_values, jnp.zeros((pad,), jnp.float32)])

  p1 = _sc_layer(all_emb, srcp, dstp, wp)
  p2 = _sc_layer(_next_table(p1), srcp, dstp, wp)
  p3 = _sc_layer(_next_table(p2), srcp, dstp, wp)
  out = _fold(all_emb, p1, p2, p3)
  return out[:NUM_USERS], out[NUM_USERS:]


# speedup vs baseline: 1.1753x; 1.0452x over previous
import functools

import jax
import jax.numpy as jnp
from jax import lax
from jax.experimental import pallas as pl
from jax.experimental.pallas import tpu as pltpu
from jax.experimental.pallas import tpu_sc as plsc

NUM_USERS = 2000
NUM_ITEMS = 8000
EMBED_DIM = 128
N_LAYERS = 3
N_NODES = NUM_USERS + NUM_ITEMS
N_EDGES = 320000

NC = 2
NS = 16
L = 16
NW = NC * NS

CHUNK = 80
CPT = 125

ROWS_PER_SUB = 624
TAIL_ROWS = N_NODES - ROWS_PER_SUB * NS


def _sc_layer(table, adj, wp):
  mesh = plsc.VectorSubcoreMesh(core_axis_name="c", subcore_axis_name="s")

  @functools.partial(
      pl.kernel,
      out_type=jax.ShapeDtypeStruct((NC, N_NODES, EMBED_DIM), jnp.float32),
      mesh=mesh,
      scratch_types=[
          pltpu.VMEM((CPT * CHUNK,), jnp.int32),
          pltpu.VMEM((CPT, CHUNK), jnp.int32),
          pltpu.VMEM((CHUNK,), jnp.float32),
          pltpu.VMEM((CHUNK,), jnp.float32),
          pltpu.VMEM((CHUNK, EMBED_DIM), jnp.float32),
          pltpu.VMEM((CHUNK, EMBED_DIM), jnp.float32),
          pltpu.VMEM_SHARED((N_NODES, EMBED_DIM), jnp.float32),
          pltpu.SemaphoreType.DMA,
          pltpu.SemaphoreType.DMA,
      ],
  )
  def k(table_h, adj_h, w_h, out_h, src_all, dst_all, w0, w1, rows0,
        rows1, acc_sh, sem0, sem1):
    c = lax.axis_index("c")
    s = lax.axis_index("s")
    wid = s * NC + c
    eb = wid * (CPT * CHUNK)

    def dpre(i, _):
      o = pl.ds(eb + i * CHUNK, CHUNK)
      v = pl.ds(i * CHUNK, CHUNK)
      pltpu.async_copy(adj_h.at[pl.ds(N_EDGES + eb + i * CHUNK, CHUNK)], src_all.at[v], sem1)
      pltpu.async_copy(adj_h.at[o], dst_all.at[i], sem1)
      return 0

    lax.fori_loop(0, CPT, dpre, 0)

    zeros16 = jnp.zeros((L,), jnp.float32)

    def zbody(i, _):
      for d in range(EMBED_DIM // L):
        rows0[i, pl.ds(d * L, L)] = zeros16
      return 0

    lax.fori_loop(0, CHUNK, zbody, 0)
    for z in range(ROWS_PER_SUB // CHUNK):
      pltpu.sync_copy(rows0,
                      acc_sh.at[pl.ds(s * ROWS_PER_SUB + z * CHUNK, CHUNK)])
    ztail = ROWS_PER_SUB - (ROWS_PER_SUB // CHUNK) * CHUNK
    if ztail:
      pltpu.sync_copy(
          rows0.at[pl.ds(0, ztail)],
          acc_sh.at[pl.ds(s * ROWS_PER_SUB + ROWS_PER_SUB - ztail, ztail)])

    @pl.when(s == 0)
    def _():
      pltpu.sync_copy(rows0.at[pl.ds(0, TAIL_ROWS)],
                      acc_sh.at[pl.ds(ROWS_PER_SUB * NS, TAIL_ROWS)])

    def ddrain(i, _):
      o = pl.ds(eb + i * CHUNK, CHUNK)
      v = pl.ds(i * CHUNK, CHUNK)
      pltpu.make_async_copy(adj_h.at[pl.ds(N_EDGES + eb + i * CHUNK, CHUNK)], src_all.at[v], sem1).wait()
      pltpu.make_async_copy(adj_h.at[o], dst_all.at[i], sem1).wait()
      return 0

    lax.fori_loop(0, CPT, ddrain, 0)
    plsc.subcore_barrier()

    rows = (rows0, rows1)
    wbufs = (w0, w1)
    gsems = (sem0, sem1)

    def gather_start(ci, b):
      pltpu.async_copy(w_h.at[pl.ds(eb + ci * CHUNK, CHUNK)], wbufs[b],
                       gsems[b])
      pltpu.async_copy(table_h.at[src_all.at[pl.ds(ci * CHUNK, CHUNK)]],
                       rows[b], gsems[b])

    def gather_wait(ci, b):
      pltpu.make_async_copy(w_h.at[pl.ds(eb + ci * CHUNK, CHUNK)], wbufs[b],
                            gsems[b]).wait()
      pltpu.make_async_copy(table_h.at[src_all.at[pl.ds(ci * CHUNK, CHUNK)]],
                            rows[b], gsems[b]).wait()

    def scale_scatter(ci, b):
      rv = rows[b]
      wv = wbufs[b]

      def sbody(g, _):
        wg = wv[pl.ds(g * L, L)]
        for j in range(L):
          e = g * L + j
          wsp = jnp.full((L,), wg[j], jnp.float32)
          for d in range(EMBED_DIM // L):
            rv[e, pl.ds(d * L, L)] = rv[e, pl.ds(d * L, L)] * wsp
        return 0

      lax.fori_loop(0, CHUNK // L, sbody, 0)
      pltpu.sync_copy(rv, acc_sh.at[dst_all.at[ci]], add=True)

    gather_start(0, 0)

    def pair(p, _):
      i0 = 2 * p
      gather_start(i0 + 1, 1)
      gather_wait(i0, 0)
      scale_scatter(i0, 0)
      nxt = jnp.minimum(i0 + 2, CPT - 1)
      gather_start(nxt, 0)
      gather_wait(i0 + 1, 1)
      scale_scatter(i0 + 1, 1)
      return 0

    lax.fori_loop(0, CPT // 2, pair, 0)
    gather_wait(CPT - 1, 0)

    plsc.subcore_barrier()
    pltpu.sync_copy(acc_sh.at[pl.ds(s * ROWS_PER_SUB, ROWS_PER_SUB)],
                    out_h.at[c, pl.ds(s * ROWS_PER_SUB, ROWS_PER_SUB)])

    @pl.when(s == 0)
    def _():
      pltpu.sync_copy(acc_sh.at[pl.ds(ROWS_PER_SUB * NS, TAIL_ROWS)],
                      out_h.at[c, pl.ds(ROWS_PER_SUB * NS, TAIL_ROWS)])

  return k(table, adj, wp)


_NB = 10
_BLK = N_NODES // _NB


def _next_table(partials):
  def body(p_ref, t_ref):
    t_ref[...] = p_ref[0] + p_ref[1]

  return pl.pallas_call(
      body,
      grid=(_NB,),
      in_specs=[pl.BlockSpec((2, _BLK, EMBED_DIM), lambda i: (0, i, 0))],
      out_specs=pl.BlockSpec((_BLK, EMBED_DIM), lambda i: (i, 0)),
      out_shape=jax.ShapeDtypeStruct((N_NODES, EMBED_DIM), jnp.float32),
  )(partials)


def _fold(e0, p1, p2, p3):
  def body(e_ref, a_ref, b_ref, c_ref, o_ref):
    o_ref[...] = (e_ref[...] + (a_ref[0] + a_ref[1]) + (b_ref[0] + b_ref[1]) +
                  (c_ref[0] + c_ref[1])) * 0.25

  pspec = pl.BlockSpec((2, _BLK, EMBED_DIM), lambda i: (0, i, 0))
  espec = pl.BlockSpec((_BLK, EMBED_DIM), lambda i: (i, 0))
  return pl.pallas_call(
      body,
      grid=(_NB,),
      in_specs=[espec, pspec, pspec, pspec],
      out_specs=espec,
      out_shape=jax.ShapeDtypeStruct((N_NODES, EMBED_DIM), jnp.float32),
  )(e0, p1, p2, p3)


def kernel(adj_indices, adj_values, user_emb, item_emb):
  all_emb = jnp.concatenate([user_emb, item_emb], axis=0)
  adj = adj_indices.astype(jnp.int32).reshape(-1)

  p1 = _sc_layer(all_emb, adj, adj_values)
  p2 = _sc_layer(_next_table(p1), adj, adj_values)
  p3 = _sc_layer(_next_table(p2), adj, adj_values)
  out = _fold(all_emb, p1, p2, p3)
  return out[:NUM_USERS], out[NUM_USERS:]
